# baseline (speedup 1.0000x reference)
import jax
import jax.numpy as jnp
from jax import lax
from jax.experimental import pallas as pl
from jax.experimental.pallas import tpu as pltpu
from jax.experimental.pallas import tpu_sc as plsc

N = 100000
D = 128
S = 512
NC = 2
NS = 16
NW = NC * NS
CHUNK = 128
TOT_CHUNKS = 800
CPW = TOT_CHUNKS // NW
FULL_CHUNKS = N // CHUNK
REM = N - FULL_CHUNKS * CHUNK
RTOT = CPW * CHUNK
RMAX = 64


def _seg_sum_body(x_hbm, ids_hbm, sums_hbm, cnts_hbm, xout_hbm,
                  idx_v, buf_a, buf_b, ones_v, cidx_v, cval_v,
                  acc_sh, cnt_sh,
                  sem_a, sem_b, sem_wa, sem_wb, sem_sa, sem_sb):
    cid = lax.axis_index("c")
    sid = lax.axis_index("s")
    w = sid * NC + cid

    zvec = jnp.zeros((16,), jnp.float32)
    onevec = jnp.ones((16,), jnp.float32)

    @pl.loop(0, 33)
    def _(r):
        for k in range(D // 16):
            buf_a[r, pl.ds(16 * k, 16)] = zvec

    @pl.loop(0, CHUNK)
    def _(r):
        for k in range(D // 16):
            ones_v[r, pl.ds(16 * k, 16)] = onevec

    pad512 = jnp.full((16,), S, jnp.int32)
    for k in range((RMAX + 16) // 16):
        cidx_v[pl.ds(16 * k, 16)] = pad512

    pltpu.sync_copy(buf_a.at[pl.ds(0, 32)], acc_sh.at[pl.ds(32 * sid, 32)])
    pltpu.sync_copy(buf_a.at[pl.ds(0, 32)], cnt_sh.at[pl.ds(32 * sid, 32)])

    @pl.when(sid == 0)
    def _():
        pltpu.sync_copy(buf_a.at[pl.ds(0, 1)], acc_sh.at[pl.ds(S, 1)])
        pltpu.sync_copy(buf_a.at[pl.ds(0, 1)], cnt_sh.at[pl.ds(S, 1)])

    pltpu.sync_copy(ids_hbm.at[w], idx_v)

    lane_iota = lax.iota(jnp.int32, 16)

    def emit(pslot, seg, runlen):
        b16 = (pslot >> 4) << 4
        grp = cidx_v[pl.ds(b16, 16)]
        cidx_v[pl.ds(b16, 16)] = jnp.where(
            lane_iota == (pslot & 15), seg, grp)
        val = jnp.broadcast_to(runlen.astype(jnp.float32), (16,))
        for k in range(D // 16):
            cval_v[pslot, pl.ds(16 * k, 16)] = val

    def scan_group(g, c):
        base = g << 4
        gr = g >> 3
        gc = (g & 7) << 4
        vec = idx_v[gr, pl.ds(gc, 16)]
        rs2, rst2, p2, ov2 = c
        def lane_min(v):
            for sh in (8, 4, 2, 1):
                shuf = v.at[lane_iota ^ sh].get(mode="promise_in_bounds")
                v = jnp.minimum(v, shuf)
            return v[0]

        for _ in range(2):
            neqb = (vec != rs2) & (lane_iota > rst2 - base)
            f = lane_min(jnp.where(neqb, lane_iota, jnp.int32(16)))
            has = f < 16
            fc = jnp.minimum(f, 15)
            pos = base + fc
            runlen = pos - rst2
            seg_new = lane_min(jnp.where(lane_iota == fc, vec,
                                         jnp.int32(1 << 20)))
            pslot = jnp.where(p2 < RMAX, p2, jnp.int32(RMAX))

            @pl.when(has)
            def _():
                emit(pslot, rs2, runlen)

            p2 = p2 + has.astype(jnp.int32)
            rs2 = jnp.where(has, seg_new, rs2)
            rst2 = jnp.where(has, pos, rst2)
        leftb = (vec != rs2) & (lane_iota > rst2 - base)
        left = lane_min(jnp.where(leftb, lane_iota, jnp.int32(16)))
        return (rs2, rst2, p2, ov2 | (left < 16).astype(jnp.int32))

    first_seg = idx_v[0, pl.ds(0, 16)][0]
    init = (first_seg, jnp.int32(0), jnp.int32(0), jnp.int32(0))

    carry = pl.loop(0, RTOT // 16, init_carry=init)(scan_group)

    run_seg, run_start, p, ovf = carry
    final_slot = jnp.where(p < RMAX, p, jnp.int32(RMAX))
    emit(final_slot, run_seg, jnp.int32(RTOT) - run_start)
    bad = (ovf > 0) | (p + 1 > RMAX)

    plsc.subcore_barrier()

    @pl.when(jnp.logical_not(bad))
    def _():
        pltpu.sync_copy(cval_v, cnt_sh.at[cidx_v], add=True)

    @pl.when(bad)
    def _():
        @pl.loop(0, CPW)
        def _(j):
            c = w * CPW + j

            @pl.when(c <= FULL_CHUNKS)
            def _():
                pltpu.sync_copy(ones_v, cnt_sh.at[idx_v.at[j]], add=True)

    def start_load(jl, buf, sem):
        c = w * CPW + jl
        row0 = c * CHUNK

        @pl.when((jl < CPW) & (c < FULL_CHUNKS))
        def _():
            pltpu.make_async_copy(x_hbm.at[pl.ds(row0, CHUNK)], buf, sem).start()

        @pl.when((jl < CPW) & (c == FULL_CHUNKS))
        def _():
            pltpu.make_async_copy(
                x_hbm.at[pl.ds(row0, REM)], buf.at[pl.ds(0, REM)], sem).start()

    def wait_load(jl, buf, sem):
        c = w * CPW + jl

        @pl.when((jl < CPW) & (c < FULL_CHUNKS))
        def _():
            pltpu.make_async_copy(x_hbm.at[pl.ds(0, CHUNK)], buf, sem).wait()

        @pl.when((jl < CPW) & (c == FULL_CHUNKS))
        def _():
            pltpu.make_async_copy(
                x_hbm.at[pl.ds(0, REM)], buf.at[pl.ds(0, REM)], sem).wait()

    def start_scat(jl, buf, sem):
        c = w * CPW + jl

        @pl.when((jl < CPW) & (c <= FULL_CHUNKS))
        def _():
            pltpu.async_copy(buf, acc_sh.at[idx_v.at[jl]], sem, add=True)

    def wait_scat(jl, buf, sem):
        c = w * CPW + jl

        @pl.when((jl < CPW) & (c <= FULL_CHUNKS))
        def _():
            pltpu.make_async_copy(buf, acc_sh.at[idx_v.at[jl]], sem).wait()

    def start_write(jl, buf, sem):
        c = w * CPW + jl
        row0 = c * CHUNK

        @pl.when((jl < CPW) & (c < FULL_CHUNKS))
        def _():
            pltpu.make_async_copy(buf, xout_hbm.at[pl.ds(row0, CHUNK)], sem).start()

        @pl.when((jl < CPW) & (c == FULL_CHUNKS))
        def _():
            pltpu.make_async_copy(
                buf.at[pl.ds(0, REM)], xout_hbm.at[pl.ds(row0, REM)], sem).start()

    def wait_write(jl, buf, sem):
        c = w * CPW + jl

        @pl.when((jl < CPW) & (c < FULL_CHUNKS))
        def _():
            pltpu.make_async_copy(buf, xout_hbm.at[pl.ds(0, CHUNK)], sem).wait()

        @pl.when((jl < CPW) & (c == FULL_CHUNKS))
        def _():
            pltpu.make_async_copy(
                buf.at[pl.ds(0, REM)], xout_hbm.at[pl.ds(0, REM)], sem).wait()

    start_load(0, buf_a, sem_a)
    start_load(1, buf_b, sem_b)

    @pl.loop(0, CPW + 1, step=2)
    def _(j):
        wait_load(j, buf_a, sem_a)
        start_write(j, buf_a, sem_wa)
        start_scat(j, buf_a, sem_sa)
        wait_load(j + 1, buf_b, sem_b)
        start_write(j + 1, buf_b, sem_wb)
        start_scat(j + 1, buf_b, sem_sb)
        wait_write(j, buf_a, sem_wa)
        wait_scat(j, buf_a, sem_sa)
        start_load(j + 2, buf_a, sem_a)
        wait_write(j + 1, buf_b, sem_wb)
        wait_scat(j + 1, buf_b, sem_sb)
        start_load(j + 3, buf_b, sem_b)

    plsc.subcore_barrier()

    pltpu.sync_copy(acc_sh.at[pl.ds(32 * sid, 32)],
                    sums_hbm.at[cid, pl.ds(32 * sid, 32)])
    pltpu.sync_copy(cnt_sh.at[pl.ds(32 * sid, 32)],
                    cnts_hbm.at[cid, pl.ds(32 * sid, 32)])


@jax.jit
def _seg_sum(x, ids3d):
    return pl.kernel(
        _seg_sum_body,
        out_type=[
            jax.ShapeDtypeStruct((NC, S, D), jnp.float32),
            jax.ShapeDtypeStruct((NC, S, D), jnp.float32),
            jax.ShapeDtypeStruct((N, D), jnp.float32),
        ],
        mesh=plsc.VectorSubcoreMesh(
            core_axis_name="c", subcore_axis_name="s",
            num_cores=NC, num_subcores=NS),
        scratch_types=[
            pltpu.VMEM((CPW, CHUNK), jnp.int32),
            pltpu.VMEM((CHUNK, D), jnp.float32),
            pltpu.VMEM((CHUNK, D), jnp.float32),
            pltpu.VMEM((CHUNK, D), jnp.float32),
            pltpu.VMEM((RMAX + 16,), jnp.int32),
            pltpu.VMEM((RMAX + 16, D), jnp.float32),
            pltpu.VMEM_SHARED((S + 1, D), jnp.float32),
            pltpu.VMEM_SHARED((S + 1, D), jnp.float32),
            pltpu.SemaphoreType.DMA,
            pltpu.SemaphoreType.DMA,
            pltpu.SemaphoreType.DMA,
            pltpu.SemaphoreType.DMA,
            pltpu.SemaphoreType.DMA,
            pltpu.SemaphoreType.DMA,
        ],
    )(x, ids3d)


def _combine_body(sums_ref, cnts_ref, out_ref):
    s = sums_ref[0] + sums_ref[1]
    c = cnts_ref[0, :, 0:1] + cnts_ref[1, :, 0:1]
    out_ref[...] = s / jnp.maximum(c, 1.0)


@jax.jit
def _combine(sums, cnts):
    return pl.pallas_call(
        _combine_body,
        out_shape=jax.ShapeDtypeStruct((S, D), jnp.float32),
    )(sums, cnts)


def kernel(t0, t1, t2, t3, t4, t5, t6):
    ids = t4.astype(jnp.int32)
    pad = jnp.full((TOT_CHUNKS * CHUNK - N,), S, dtype=jnp.int32)
    ids3d = jnp.concatenate([ids, pad]).reshape(NW, CPW, CHUNK)
    sums, cnts, x_out = _seg_sum(t0, ids3d)
    x_graph = _combine(sums, cnts)
    return (x_out, t1, t2, t3, t4, x_graph, t6)


# speedup vs baseline: 1.1173x; 1.1173x over previous
import jax
import jax.numpy as jnp
from jax import lax
from jax.experimental import pallas as pl
from jax.experimental.pallas import tpu as pltpu
from jax.experimental.pallas import tpu_sc as plsc

N = 100000
D = 128
S = 512
NC = 2
NS = 16
NW = NC * NS
CHUNK = 128
TOT_CHUNKS = 800
CPW = TOT_CHUNKS // NW
FULL_CHUNKS = N // CHUNK
REM = N - FULL_CHUNKS * CHUNK
RTOT = CPW * CHUNK
RMAX = 64


def _seg_sum_body(x_hbm, ids_hbm, sums_hbm, cnts_hbm, xout_hbm,
                  idx_v, buf_a, buf_b, ones_v, cidx_v, cval_v, scal_s,
                  acc_sh, cnt_sh,
                  sem_a, sem_b, sem_wa, sem_wb, sem_sa, sem_sb):
    cid = lax.axis_index("c")
    sid = lax.axis_index("s")
    w = sid * NC + cid

    zvec = jnp.zeros((16,), jnp.float32)
    onevec = jnp.ones((16,), jnp.float32)

    @pl.loop(0, 33)
    def _(r):
        for k in range(D // 16):
            buf_a[r, pl.ds(16 * k, 16)] = zvec

    @pl.loop(0, CHUNK)
    def _(r):
        for k in range(D // 16):
            ones_v[r, pl.ds(16 * k, 16)] = onevec

    pad512 = jnp.full((16,), S, jnp.int32)
    for k in range((RMAX + 16) // 16):
        cidx_v[pl.ds(16 * k, 16)] = pad512

    pltpu.sync_copy(buf_a.at[pl.ds(0, 32)], acc_sh.at[pl.ds(32 * sid, 32)])
    pltpu.sync_copy(buf_a.at[pl.ds(0, 32)], cnt_sh.at[pl.ds(32 * sid, 32)])

    @pl.when(sid == 0)
    def _():
        pltpu.sync_copy(buf_a.at[pl.ds(0, 1)], acc_sh.at[pl.ds(S, 1)])
        pltpu.sync_copy(buf_a.at[pl.ds(0, 1)], cnt_sh.at[pl.ds(S, 1)])

    pltpu.sync_copy(ids_hbm.at[w], idx_v)

    lane_iota = lax.iota(jnp.int32, 16)

    def emit(pslot, seg, runlen):
        b16 = (pslot >> 4) << 4
        grp = cidx_v[pl.ds(b16, 16)]
        cidx_v[pl.ds(b16, 16)] = jnp.where(
            lane_iota == (pslot & 15), seg, grp)
        val = jnp.broadcast_to(runlen.astype(jnp.float32), (16,))
        for k in range(D // 16):
            cval_v[pslot, pl.ds(16 * k, 16)] = val

    def lane_min(v):
        for sh in (8, 4, 2, 1):
            shuf = v.at[lane_iota ^ sh].get(mode="promise_in_bounds")
            v = jnp.minimum(v, shuf)
        return v[0]

    first_seg = idx_v[0, pl.ds(0, 16)][0]
    scal_s[0] = first_seg
    scal_s[1] = jnp.int32(0)
    scal_s[2] = jnp.int32(0)
    scal_s[3] = jnp.int32(0)

    @pl.loop(0, RTOT // 16)
    def _(g):
        base = g << 4
        vec = idx_v[g >> 3, pl.ds((g & 7) << 4, 16)]

        @pl.when(vec[15] != scal_s[0])
        def _():
            rs2 = scal_s[0]
            rst2 = scal_s[1]
            p2 = scal_s[2]
            ov2 = scal_s[3]
            for _ in range(2):
                neqb = (vec != rs2) & (lane_iota > rst2 - base)
                f = lane_min(jnp.where(neqb, lane_iota, jnp.int32(16)))
                has = f < 16
                fc = jnp.minimum(f, 15)
                pos = base + fc
                runlen = pos - rst2
                seg_new = lane_min(jnp.where(lane_iota == fc, vec,
                                             jnp.int32(1 << 20)))
                pslot = jnp.where(p2 < RMAX, p2, jnp.int32(RMAX))

                @pl.when(has)
                def _():
                    emit(pslot, rs2, runlen)

                p2 = p2 + has.astype(jnp.int32)
                rs2 = jnp.where(has, seg_new, rs2)
                rst2 = jnp.where(has, pos, rst2)
            leftb = (vec != rs2) & (lane_iota > rst2 - base)
            left = lane_min(jnp.where(leftb, lane_iota, jnp.int32(16)))
            scal_s[0] = rs2
            scal_s[1] = rst2
            scal_s[2] = p2
            scal_s[3] = ov2 | (left < 16).astype(jnp.int32)

    run_seg = scal_s[0]
    run_start = scal_s[1]
    p = scal_s[2]
    ovf = scal_s[3]
    final_slot = jnp.where(p < RMAX, p, jnp.int32(RMAX))
    emit(final_slot, run_seg, jnp.int32(RTOT) - run_start)
    bad = (ovf > 0) | (p + 1 > RMAX)

    plsc.subcore_barrier()

    @pl.when(jnp.logical_not(bad))
    def _():
        pltpu.sync_copy(cval_v, cnt_sh.at[cidx_v], add=True)

    @pl.when(bad)
    def _():
        @pl.loop(0, CPW)
        def _(j):
            c = w * CPW + j

            @pl.when(c <= FULL_CHUNKS)
            def _():
                pltpu.sync_copy(ones_v, cnt_sh.at[idx_v.at[j]], add=True)

    def start_load(jl, buf, sem):
        c = w * CPW + jl
        row0 = c * CHUNK

        @pl.when((jl < CPW) & (c < FULL_CHUNKS))
        def _():
            pltpu.make_async_copy(x_hbm.at[pl.ds(row0, CHUNK)], buf, sem).start()

        @pl.when((jl < CPW) & (c == FULL_CHUNKS))
        def _():
            pltpu.make_async_copy(
                x_hbm.at[pl.ds(row0, REM)], buf.at[pl.ds(0, REM)], sem).start()

    def wait_load(jl, buf, sem):
        c = w * CPW + jl

        @pl.when((jl < CPW) & (c < FULL_CHUNKS))
        def _():
            pltpu.make_async_copy(x_hbm.at[pl.ds(0, CHUNK)], buf, sem).wait()

        @pl.when((jl < CPW) & (c == FULL_CHUNKS))
        def _():
            pltpu.make_async_copy(
                x_hbm.at[pl.ds(0, REM)], buf.at[pl.ds(0, REM)], sem).wait()

    def start_scat(jl, buf, sem):
        c = w * CPW + jl

        @pl.when((jl < CPW) & (c <= FULL_CHUNKS))
        def _():
            pltpu.async_copy(buf, acc_sh.at[idx_v.at[jl]], sem, add=True)

    def wait_scat(jl, buf, sem):
        c = w * CPW + jl

        @pl.when((jl < CPW) & (c <= FULL_CHUNKS))
        def _():
            pltpu.make_async_copy(buf, acc_sh.at[idx_v.at[jl]], sem).wait()

    def start_write(jl, buf, sem):
        c = w * CPW + jl
        row0 = c * CHUNK

        @pl.when((jl < CPW) & (c < FULL_CHUNKS))
        def _():
            pltpu.make_async_copy(buf, xout_hbm.at[pl.ds(row0, CHUNK)], sem).start()

        @pl.when((jl < CPW) & (c == FULL_CHUNKS))
        def _():
            pltpu.make_async_copy(
                buf.at[pl.ds(0, REM)], xout_hbm.at[pl.ds(row0, REM)], sem).start()

    def wait_write(jl, buf, sem):
        c = w * CPW + jl

        @pl.when((jl < CPW) & (c < FULL_CHUNKS))
        def _():
            pltpu.make_async_copy(buf, xout_hbm.at[pl.ds(0, CHUNK)], sem).wait()

        @pl.when((jl < CPW) & (c == FULL_CHUNKS))
        def _():
            pltpu.make_async_copy(
                buf.at[pl.ds(0, REM)], xout_hbm.at[pl.ds(0, REM)], sem).wait()

    start_load(0, buf_a, sem_a)
    start_load(1, buf_b, sem_b)

    @pl.loop(0, CPW + 1, step=2)
    def _(j):
        wait_load(j, buf_a, sem_a)
        start_write(j, buf_a, sem_wa)
        start_scat(j, buf_a, sem_sa)
        wait_load(j + 1, buf_b, sem_b)
        start_write(j + 1, buf_b, sem_wb)
        start_scat(j + 1, buf_b, sem_sb)
        wait_write(j, buf_a, sem_wa)
        wait_scat(j, buf_a, sem_sa)
        start_load(j + 2, buf_a, sem_a)
        wait_write(j + 1, buf_b, sem_wb)
        wait_scat(j + 1, buf_b, sem_sb)
        start_load(j + 3, buf_b, sem_b)

    plsc.subcore_barrier()

    pltpu.sync_copy(acc_sh.at[pl.ds(32 * sid, 32)],
                    sums_hbm.at[cid, pl.ds(32 * sid, 32)])
    pltpu.sync_copy(cnt_sh.at[pl.ds(32 * sid, 32)],
                    cnts_hbm.at[cid, pl.ds(32 * sid, 32)])


@jax.jit
def _seg_sum(x, ids3d):
    return pl.kernel(
        _seg_sum_body,
        out_type=[
            jax.ShapeDtypeStruct((NC, S, D), jnp.float32),
            jax.ShapeDtypeStruct((NC, S, D), jnp.float32),
            jax.ShapeDtypeStruct((N, D), jnp.float32),
        ],
        mesh=plsc.VectorSubcoreMesh(
            core_axis_name="c", subcore_axis_name="s",
            num_cores=NC, num_subcores=NS),
        scratch_types=[
            pltpu.VMEM((CPW, CHUNK), jnp.int32),
            pltpu.VMEM((CHUNK, D), jnp.float32),
            pltpu.VMEM((CHUNK, D), jnp.float32),
            pltpu.VMEM((CHUNK, D), jnp.float32),
            pltpu.VMEM((RMAX + 16,), jnp.int32),
            pltpu.VMEM((RMAX + 16, D), jnp.float32),
            pltpu.SMEM((8,), jnp.int32),
            pltpu.VMEM_SHARED((S + 1, D), jnp.float32),
            pltpu.VMEM_SHARED((S + 1, D), jnp.float32),
            pltpu.SemaphoreType.DMA,
            pltpu.SemaphoreType.DMA,
            pltpu.SemaphoreType.DMA,
            pltpu.SemaphoreType.DMA,
            pltpu.SemaphoreType.DMA,
            pltpu.SemaphoreType.DMA,
        ],
    )(x, ids3d)


def _combine_body(sums_ref, cnts_ref, out_ref):
    s = sums_ref[0] + sums_ref[1]
    c = cnts_ref[0, :, 0:1] + cnts_ref[1, :, 0:1]
    out_ref[...] = s / jnp.maximum(c, 1.0)


@jax.jit
def _combine(sums, cnts):
    return pl.pallas_call(
        _combine_body,
        out_shape=jax.ShapeDtypeStruct((S, D), jnp.float32),
    )(sums, cnts)


def kernel(t0, t1, t2, t3, t4, t5, t6):
    ids = t4.astype(jnp.int32)
    pad = jnp.full((TOT_CHUNKS * CHUNK - N,), S, dtype=jnp.int32)
    ids3d = jnp.concatenate([ids, pad]).reshape(NW, CPW, CHUNK)
    sums, cnts, x_out = _seg_sum(t0, ids3d)
    x_graph = _combine(sums, cnts)
    return (x_out, t1, t2, t3, t4, x_graph, t6)


# speedup vs baseline: 1.2216x; 1.0934x over previous
import jax
import jax.numpy as jnp
from jax import lax
from jax.experimental import pallas as pl
from jax.experimental.pallas import tpu as pltpu
from jax.experimental.pallas import tpu_sc as plsc

N = 100000
D = 128
S = 512
NC = 2
NS = 16
NW = NC * NS
CHUNK = 128
TOT_CHUNKS = 800
CPW = TOT_CHUNKS // NW
FULL_CHUNKS = N // CHUNK
REM = N - FULL_CHUNKS * CHUNK


def _seg_sum_body(x_hbm, ids_hbm, sums_hbm, cnts_hbm, xout_hbm,
                  idx_v, buf_a, buf_b, ones_v, acc_sh, cnt_sh,
                  sem_a, sem_b, sem_wa, sem_wb, sem_sa, sem_sb, sem_c):
    cid = lax.axis_index("c")
    sid = lax.axis_index("s")
    w = sid * NC + cid

    zvec = jnp.zeros((16,), jnp.float32)
    onevec = jnp.ones((16,), jnp.float32)

    @pl.loop(0, 33)
    def _(r):
        for k in range(D // 16):
            buf_a[r, pl.ds(16 * k, 16)] = zvec

    @pl.loop(0, CHUNK)
    def _(r):
        for k in range(D // 16):
            ones_v[r, pl.ds(16 * k, 16)] = onevec

    pltpu.sync_copy(buf_a.at[pl.ds(0, 32)], acc_sh.at[pl.ds(32 * sid, 32)])
    pltpu.sync_copy(buf_a.at[pl.ds(0, 32)], cnt_sh.at[pl.ds(32 * sid, 32)])

    @pl.when(sid == 0)
    def _():
        pltpu.sync_copy(buf_a.at[pl.ds(0, 1)], acc_sh.at[pl.ds(S, 1)])
        pltpu.sync_copy(buf_a.at[pl.ds(0, 1)], cnt_sh.at[pl.ds(S, 1)])

    pltpu.sync_copy(ids_hbm.at[w], idx_v)

    plsc.subcore_barrier()

    def start_load(jl, buf, sem):
        c = w * CPW + jl
        row0 = c * CHUNK

        @pl.when((jl < CPW) & (c < FULL_CHUNKS))
        def _():
            pltpu.make_async_copy(x_hbm.at[pl.ds(row0, CHUNK)], buf, sem).start()

        @pl.when((jl < CPW) & (c == FULL_CHUNKS))
        def _():
            pltpu.make_async_copy(
                x_hbm.at[pl.ds(row0, REM)], buf.at[pl.ds(0, REM)], sem).start()

    def wait_load(jl, buf, sem):
        c = w * CPW + jl

        @pl.when((jl < CPW) & (c < FULL_CHUNKS))
        def _():
            pltpu.make_async_copy(x_hbm.at[pl.ds(0, CHUNK)], buf, sem).wait()

        @pl.when((jl < CPW) & (c == FULL_CHUNKS))
        def _():
            pltpu.make_async_copy(
                x_hbm.at[pl.ds(0, REM)], buf.at[pl.ds(0, REM)], sem).wait()

    def start_scat(jl, buf, sem):
        c = w * CPW + jl

        @pl.when((jl < CPW) & (c <= FULL_CHUNKS))
        def _():
            pltpu.async_copy(buf, acc_sh.at[idx_v.at[jl]], sem, add=True)

    def wait_scat(jl, buf, sem):
        c = w * CPW + jl

        @pl.when((jl < CPW) & (c <= FULL_CHUNKS))
        def _():
            pltpu.make_async_copy(buf, acc_sh.at[idx_v.at[jl]], sem).wait()

    def start_cnt(jl, sem):
        c = w * CPW + jl

        @pl.when((jl < CPW) & (c <= FULL_CHUNKS))
        def _():
            pltpu.async_copy(ones_v, cnt_sh.at[idx_v.at[jl]], sem, add=True)

    def wait_cnt(jl, sem):
        c = w * CPW + jl

        @pl.when((jl < CPW) & (c <= FULL_CHUNKS))
        def _():
            pltpu.make_async_copy(ones_v, cnt_sh.at[idx_v.at[jl]], sem).wait()

    def start_write(jl, buf, sem):
        c = w * CPW + jl
        row0 = c * CHUNK

        @pl.when((jl < CPW) & (c < FULL_CHUNKS))
        def _():
            pltpu.make_async_copy(buf, xout_hbm.at[pl.ds(row0, CHUNK)], sem).start()

        @pl.when((jl < CPW) & (c == FULL_CHUNKS))
        def _():
            pltpu.make_async_copy(
                buf.at[pl.ds(0, REM)], xout_hbm.at[pl.ds(row0, REM)], sem).start()

    def wait_write(jl, buf, sem):
        c = w * CPW + jl

        @pl.when((jl < CPW) & (c < FULL_CHUNKS))
        def _():
            pltpu.make_async_copy(buf, xout_hbm.at[pl.ds(0, CHUNK)], sem).wait()

        @pl.when((jl < CPW) & (c == FULL_CHUNKS))
        def _():
            pltpu.make_async_copy(
                buf.at[pl.ds(0, REM)], xout_hbm.at[pl.ds(0, REM)], sem).wait()

    start_load(0, buf_a, sem_a)
    start_load(1, buf_b, sem_b)

    @pl.loop(0, CPW + 1, step=2)
    def _(j):
        wait_load(j, buf_a, sem_a)
        start_write(j, buf_a, sem_wa)
        start_scat(j, buf_a, sem_sa)
        start_cnt(j, sem_c)
        wait_load(j + 1, buf_b, sem_b)
        start_write(j + 1, buf_b, sem_wb)
        start_scat(j + 1, buf_b, sem_sb)
        start_cnt(j + 1, sem_c)
        wait_write(j, buf_a, sem_wa)
        wait_scat(j, buf_a, sem_sa)
        start_load(j + 2, buf_a, sem_a)
        wait_write(j + 1, buf_b, sem_wb)
        wait_scat(j + 1, buf_b, sem_sb)
        start_load(j + 3, buf_b, sem_b)

    @pl.loop(0, CPW)
    def _(j):
        wait_cnt(j, sem_c)

    plsc.subcore_barrier()

    pltpu.sync_copy(acc_sh.at[pl.ds(32 * sid, 32)],
                    sums_hbm.at[cid, pl.ds(32 * sid, 32)])
    pltpu.sync_copy(cnt_sh.at[pl.ds(32 * sid, 32)],
                    cnts_hbm.at[cid, pl.ds(32 * sid, 32)])


@jax.jit
def _seg_sum(x, ids3d):
    return pl.kernel(
        _seg_sum_body,
        out_type=[
            jax.ShapeDtypeStruct((NC, S, D), jnp.float32),
            jax.ShapeDtypeStruct((NC, S, D), jnp.float32),
            jax.ShapeDtypeStruct((N, D), jnp.float32),
        ],
        mesh=plsc.VectorSubcoreMesh(
            core_axis_name="c", subcore_axis_name="s",
            num_cores=NC, num_subcores=NS),
        scratch_types=[
            pltpu.VMEM((CPW, CHUNK), jnp.int32),
            pltpu.VMEM((CHUNK, D), jnp.float32),
            pltpu.VMEM((CHUNK, D), jnp.float32),
            pltpu.VMEM((CHUNK, D), jnp.float32),
            pltpu.VMEM_SHARED((S + 1, D), jnp.float32),
            pltpu.VMEM_SHARED((S + 1, D), jnp.float32),
            pltpu.SemaphoreType.DMA,
            pltpu.SemaphoreType.DMA,
            pltpu.SemaphoreType.DMA,
            pltpu.SemaphoreType.DMA,
            pltpu.SemaphoreType.DMA,
            pltpu.SemaphoreType.DMA,
            pltpu.SemaphoreType.DMA,
        ],
    )(x, ids3d)


def _combine_body(sums_ref, cnts_ref, out_ref):
    s = sums_ref[0] + sums_ref[1]
    c = cnts_ref[0, :, 0:1] + cnts_ref[1, :, 0:1]
    out_ref[...] = s / jnp.maximum(c, 1.0)


@jax.jit
def _combine(sums, cnts):
    return pl.pallas_call(
        _combine_body,
        out_shape=jax.ShapeDtypeStruct((S, D), jnp.float32),
    )(sums, cnts)


def kernel(t0, t1, t2, t3, t4, t5, t6):
    ids = t4.astype(jnp.int32)
    pad = jnp.full((TOT_CHUNKS * CHUNK - N,), S, dtype=jnp.int32)
    ids3d = jnp.concatenate([ids, pad]).reshape(NW, CPW, CHUNK)
    sums, cnts, x_out = _seg_sum(t0, ids3d)
    x_graph = _combine(sums, cnts)
    return (x_out, t1, t2, t3, t4, x_graph, t6)
